# Initial kernel scaffold; baseline (speedup 1.0000x reference)
import functools

import jax
import jax.numpy as jnp
from jax import lax
from jax.experimental import pallas as pl
from jax.experimental.pallas import tpu as pltpu
from jax.experimental.pallas import tpu_sc as plsc

N = 10000
E = 320000
D_IN = 128
H1 = 128
H2 = 64

NC = 2
NS = 16
NW = NC * NS
CHUNK = 128
ET = 10112
NCHUNK = ET // CHUNK
EP = ET * NW
ROWS_PER_TILE = N // NS
ZROWS = 126
ACC_ROWS = NS * 5 * ZROWS

_mesh = plsc.VectorSubcoreMesh(
    core_axis_name="c", subcore_axis_name="s", num_cores=NC, num_subcores=NS
)


def _zero_vmem(ref, rows, width):
    zv = jnp.zeros((16,), jnp.float32)
    lanes = width // 16

    def body(k, _):
        i = k // lanes
        j = k % lanes
        ref[i, pl.ds(j * 16, 16)] = zv
        return 0

    lax.fori_loop(0, rows * lanes, body, 0)


def _agg_body(width, g_hbm, src_hbm, dst_hbm, out_hbm,
              srcv, dstv, rows_a, zbuf, acc, sem_a):
    c = lax.axis_index("c")
    s = lax.axis_index("s")
    t = c * NS + s

    _zero_vmem(zbuf, ZROWS, width)
    base = s * 5 * ZROWS
    for i in range(5):
        pltpu.sync_copy(zbuf, acc.at[pl.ds(base + i * ZROWS, ZROWS)])

    pltpu.sync_copy(src_hbm.at[t], srcv)
    pltpu.sync_copy(dst_hbm.at[t], dstv)

    plsc.subcore_barrier()

    def chunk(j, _):
        pltpu.async_copy(g_hbm.at[srcv.at[j]], rows_a, sem_a).wait()
        pltpu.sync_copy(rows_a, acc.at[dstv.at[j]], add=True)
        return 0

    lax.fori_loop(0, NCHUNK, chunk, 0)

    plsc.subcore_barrier()

    r0 = s * ROWS_PER_TILE
    pltpu.sync_copy(acc.at[pl.ds(r0, ROWS_PER_TILE)],
                    out_hbm.at[c].at[pl.ds(r0, ROWS_PER_TILE)])


def _make_agg(width):
    return pl.kernel(
        functools.partial(_agg_body, width),
        out_type=jax.ShapeDtypeStruct((NC, N, width), jnp.float32),
        mesh=_mesh,
        scratch_types=[
            pltpu.VMEM((NCHUNK, CHUNK), jnp.int32),
            pltpu.VMEM((NCHUNK, CHUNK), jnp.int32),
            pltpu.VMEM((CHUNK, width), jnp.float32),
            pltpu.VMEM((ZROWS, width), jnp.float32),
            pltpu.VMEM_SHARED((ACC_ROWS, width), jnp.float32),
            pltpu.SemaphoreType.DMA,
        ],
        name=f"gcn_agg_{width}",
    )


def _deg_body(dst_hbm, out_hbm, dstv, ones_b, zbuf, acc, sem):
    c = lax.axis_index("c")
    s = lax.axis_index("s")
    t = c * NS + s

    _zero_vmem(zbuf, ZROWS, 16)
    base = s * 5 * ZROWS
    for i in range(5):
        pltpu.sync_copy(zbuf, acc.at[pl.ds(base + i * ZROWS, ZROWS)])

    ov = jnp.ones((16,), jnp.float32)

    def fill(k, _):
        ones_b[k, pl.ds(0, 16)] = ov
        return 0

    lax.fori_loop(0, CHUNK, fill, 0)

    pltpu.sync_copy(dst_hbm.at[t], dstv)

    plsc.subcore_barrier()

    def chunk(j, _):
        pltpu.sync_copy(ones_b, acc.at[dstv.at[j]], add=True)
        return 0

    lax.fori_loop(0, NCHUNK, chunk, 0)

    plsc.subcore_barrier()

    r0 = s * ROWS_PER_TILE
    pltpu.sync_copy(acc.at[pl.ds(r0, ROWS_PER_TILE)],
                    out_hbm.at[c].at[pl.ds(r0, ROWS_PER_TILE)])


_deg_kernel = pl.kernel(
    _deg_body,
    out_type=jax.ShapeDtypeStruct((NC, N, 16), jnp.float32),
    mesh=_mesh,
    scratch_types=[
        pltpu.VMEM((NCHUNK, CHUNK), jnp.int32),
        pltpu.VMEM((CHUNK, 16), jnp.float32),
        pltpu.VMEM((ZROWS, 16), jnp.float32),
        pltpu.VMEM_SHARED((ACC_ROWS, 16), jnp.float32),
        pltpu.SemaphoreType.DMA,
    ],
    name="gcn_deg",
)


BN = 1000


def _tc1_body(x_ref, w_ref, d0_ref, d1_ref, g_ref):
    deg = d0_ref[:, 0:1] + d1_ref[:, 0:1] + 1.0
    dinv = lax.rsqrt(deg)
    h = jnp.dot(x_ref[...], w_ref[...], preferred_element_type=jnp.float32)
    g_ref[...] = h * dinv


def _tc2_body(a0_ref, a1_ref, g1_ref, d0_ref, d1_ref, w_ref, b_ref, g2_ref):
    deg = d0_ref[:, 0:1] + d1_ref[:, 0:1] + 1.0
    dinv = lax.rsqrt(deg)
    h1 = jnp.maximum(dinv * (a0_ref[...] + a1_ref[...] + g1_ref[...])
                     + b_ref[...], 0.0)
    h2 = jnp.dot(h1, w_ref[...], preferred_element_type=jnp.float32)
    g2_ref[...] = h2 * dinv


def _tc3_body(a0_ref, a1_ref, g2_ref, d0_ref, d1_ref, w_ref, b_ref, out_ref):
    deg = d0_ref[:, 0:1] + d1_ref[:, 0:1] + 1.0
    dinv = lax.rsqrt(deg)
    h2 = jnp.maximum(dinv * (a0_ref[...] + a1_ref[...] + g2_ref[...])
                     + b_ref[...], 0.0)
    out_ref[...] = jnp.sum(h2 * w_ref[...], axis=1)


def _row_spec(width):
    return pl.BlockSpec((BN, width), lambda i: (i, 0))


def _full_spec(a, b):
    return pl.BlockSpec((a, b), lambda i: (0, 0))


def kernel(x, edge_index, W1, b1, W2, b2, Wfc, bfc):
    src = edge_index[0]
    dst = edge_index[1]
    pad = EP - E
    srcp = jnp.concatenate([src, jnp.zeros((pad,), jnp.int32)])
    dstp = jnp.concatenate([dst, jnp.full((pad,), N, jnp.int32)])
    srcp = srcp.reshape(NW, NCHUNK, CHUNK)
    dstp = dstp.reshape(NW, NCHUNK, CHUNK)

    degp = _deg_kernel(dstp)
    d0 = degp[0, :, :8]
    d1 = degp[1, :, :8]

    grid = (N // BN,)

    g1 = pl.pallas_call(
        _tc1_body,
        grid=grid,
        in_specs=[
            _row_spec(D_IN),
            _full_spec(D_IN, H1),
            _row_spec(8),
            _row_spec(8),
        ],
        out_specs=_row_spec(H1),
        out_shape=jax.ShapeDtypeStruct((N, H1), jnp.float32),
    )(x, W1, d0, d1)

    agg1 = _make_agg(H1)(g1, srcp, dstp)

    g2 = pl.pallas_call(
        _tc2_body,
        grid=grid,
        in_specs=[
            _row_spec(H1),
            _row_spec(H1),
            _row_spec(H1),
            _row_spec(8),
            _row_spec(8),
            _full_spec(H1, H2),
            _full_spec(1, H1),
        ],
        out_specs=_row_spec(H2),
        out_shape=jax.ShapeDtypeStruct((N, H2), jnp.float32),
    )(agg1[0], agg1[1], g1, d0, d1, W2, b1.reshape(1, H1))

    agg2 = _make_agg(H2)(g2, srcp, dstp)

    out = pl.pallas_call(
        _tc3_body,
        grid=grid,
        in_specs=[
            _row_spec(H2),
            _row_spec(H2),
            _row_spec(H2),
            _row_spec(8),
            _row_spec(8),
            _full_spec(1, H2),
            _full_spec(1, H2),
        ],
        out_specs=pl.BlockSpec((BN,), lambda i: (i,)),
        out_shape=jax.ShapeDtypeStruct((N,), jnp.float32),
    )(agg2[0], agg2[1], g2, d0, d1, Wfc.reshape(1, H2), b2.reshape(1, H2))

    return out + bfc[0]


# speedup vs baseline: 13.9453x; 13.9453x over previous
import functools

import jax
import jax.numpy as jnp
from jax import lax
from jax.experimental import pallas as pl
from jax.experimental.pallas import tpu as pltpu
from jax.experimental.pallas import tpu_sc as plsc

N = 10000
E = 320000
D_IN = 128
H1 = 128
H2 = 64

NC = 2
NS = 16
NW = NC * NS
CHUNK = 128
ET = 10112
NCHUNK = ET // CHUNK
EP = ET * NW
ZROWS = 128
ROWS_PER_TILE = 5 * ZROWS
NOUT = NS * ROWS_PER_TILE
ACC_ROWS = NOUT

_mesh = plsc.VectorSubcoreMesh(
    core_axis_name="c", subcore_axis_name="s", num_cores=NC, num_subcores=NS
)


def _zero_vmem(ref, rows, width):
    zv = jnp.zeros((16,), jnp.float32)
    lanes = width // 16

    def body(k, _):
        i = k // lanes
        j = k % lanes
        ref[i, pl.ds(j * 16, 16)] = zv
        return 0

    lax.fori_loop(0, rows * lanes, body, 0)


def _agg_body(width, g_hbm, src_hbm, dst_hbm, out_hbm,
              srcv, dstv, rows_a, acc, sem_a):
    c = lax.axis_index("c")
    s = lax.axis_index("s")
    t = c * NS + s

    _zero_vmem(rows_a, ZROWS, width)
    base = s * ROWS_PER_TILE
    for i in range(5):
        pltpu.sync_copy(rows_a, acc.at[pl.ds(base + i * ZROWS, ZROWS)])

    pltpu.sync_copy(src_hbm.at[t], srcv)
    pltpu.sync_copy(dst_hbm.at[t], dstv)

    plsc.subcore_barrier()

    def chunk(j, _):
        pltpu.async_copy(g_hbm.at[srcv.at[j]], rows_a, sem_a).wait()
        pltpu.sync_copy(rows_a, acc.at[dstv.at[j]], add=True)
        return 0

    lax.fori_loop(0, NCHUNK, chunk, 0)

    plsc.subcore_barrier()

    r0 = s * ROWS_PER_TILE
    pltpu.sync_copy(acc.at[pl.ds(r0, ROWS_PER_TILE)],
                    out_hbm.at[c].at[pl.ds(r0, ROWS_PER_TILE)])


def _make_agg(width):
    return pl.kernel(
        functools.partial(_agg_body, width),
        out_type=jax.ShapeDtypeStruct((NC, NOUT, width), jnp.float32),
        mesh=_mesh,
        scratch_types=[
            pltpu.VMEM((NCHUNK, CHUNK), jnp.int32),
            pltpu.VMEM((NCHUNK, CHUNK), jnp.int32),
            pltpu.VMEM((CHUNK, width), jnp.float32),
            pltpu.VMEM_SHARED((ACC_ROWS, width), jnp.float32),
            pltpu.SemaphoreType.DMA,
        ],
        compiler_params=pltpu.CompilerParams(use_tc_tiling_on_sc=False),
        name=f"gcn_agg_{width}",
    )


def _deg_body(dst_hbm, out_hbm, dstv, ones_b, zbuf, acc, sem):
    c = lax.axis_index("c")
    s = lax.axis_index("s")
    t = c * NS + s

    _zero_vmem(zbuf, ZROWS, 16)
    base = s * ROWS_PER_TILE
    for i in range(5):
        pltpu.sync_copy(zbuf, acc.at[pl.ds(base + i * ZROWS, ZROWS)])

    ov = jnp.ones((16,), jnp.float32)

    def fill(k, _):
        ones_b[k, pl.ds(0, 16)] = ov
        return 0

    lax.fori_loop(0, CHUNK, fill, 0)

    pltpu.sync_copy(dst_hbm.at[t], dstv)

    plsc.subcore_barrier()

    def chunk(j, _):
        pltpu.sync_copy(ones_b, acc.at[dstv.at[j]], add=True)
        return 0

    lax.fori_loop(0, NCHUNK, chunk, 0)

    plsc.subcore_barrier()

    r0 = s * ROWS_PER_TILE
    pltpu.sync_copy(acc.at[pl.ds(r0, ROWS_PER_TILE)],
                    out_hbm.at[c].at[pl.ds(r0, ROWS_PER_TILE)])


_deg_kernel = pl.kernel(
    _deg_body,
    out_type=jax.ShapeDtypeStruct((NC, NOUT, 16), jnp.float32),
    mesh=_mesh,
    scratch_types=[
        pltpu.VMEM((NCHUNK, CHUNK), jnp.int32),
        pltpu.VMEM((CHUNK, 16), jnp.float32),
        pltpu.VMEM((ZROWS, 16), jnp.float32),
        pltpu.VMEM_SHARED((ACC_ROWS, 16), jnp.float32),
        pltpu.SemaphoreType.DMA,
    ],
    compiler_params=pltpu.CompilerParams(use_tc_tiling_on_sc=False),
    name="gcn_deg",
)


BN = 1000


def _tc1_body(x_ref, w_ref, d0_ref, d1_ref, g_ref):
    deg = d0_ref[:, 0:1] + d1_ref[:, 0:1] + 1.0
    dinv = lax.rsqrt(deg)
    h = jnp.dot(x_ref[...], w_ref[...], preferred_element_type=jnp.float32)
    g_ref[...] = h * dinv


def _tc2_body(a0_ref, a1_ref, g1_ref, d0_ref, d1_ref, w_ref, b_ref, g2_ref):
    deg = d0_ref[:, 0:1] + d1_ref[:, 0:1] + 1.0
    dinv = lax.rsqrt(deg)
    h1 = jnp.maximum(dinv * (a0_ref[...] + a1_ref[...] + g1_ref[...])
                     + b_ref[...], 0.0)
    h2 = jnp.dot(h1, w_ref[...], preferred_element_type=jnp.float32)
    g2_ref[...] = h2 * dinv


def _tc3_body(a0_ref, a1_ref, g2_ref, d0_ref, d1_ref, w_ref, b_ref, out_ref):
    deg = d0_ref[:, 0:1] + d1_ref[:, 0:1] + 1.0
    dinv = lax.rsqrt(deg)
    h2 = jnp.maximum(dinv * (a0_ref[...] + a1_ref[...] + g2_ref[...])
                     + b_ref[...], 0.0)
    red = jnp.sum(h2 * w_ref[...], axis=1, keepdims=True)
    out_ref[...] = jnp.broadcast_to(red, out_ref.shape)


def _row_spec(width):
    return pl.BlockSpec((BN, width), lambda i: (i, 0))


def _full_spec(a, b):
    return pl.BlockSpec((a, b), lambda i: (0, 0))


def kernel(x, edge_index, W1, b1, W2, b2, Wfc, bfc):
    src = edge_index[0]
    dst = edge_index[1]
    pad = EP - E
    srcp = jnp.concatenate([src, jnp.zeros((pad,), jnp.int32)])
    dstp = jnp.concatenate([dst, jnp.full((pad,), N, jnp.int32)])
    srcp = srcp.reshape(NW, NCHUNK, CHUNK)
    dstp = dstp.reshape(NW, NCHUNK, CHUNK)

    degp = _deg_kernel(dstp)
    d0 = degp[0, :N, :8]
    d1 = degp[1, :N, :8]

    grid = (N // BN,)

    g1 = pl.pallas_call(
        _tc1_body,
        grid=grid,
        in_specs=[
            _row_spec(D_IN),
            _full_spec(D_IN, H1),
            _row_spec(8),
            _row_spec(8),
        ],
        out_specs=_row_spec(H1),
        out_shape=jax.ShapeDtypeStruct((N, H1), jnp.float32),
    )(x, W1, d0, d1)

    agg1 = _make_agg(H1)(g1, srcp, dstp)[:, :N]

    g2 = pl.pallas_call(
        _tc2_body,
        grid=grid,
        in_specs=[
            _row_spec(H1),
            _row_spec(H1),
            _row_spec(H1),
            _row_spec(8),
            _row_spec(8),
            _full_spec(H1, H2),
            _full_spec(1, H1),
        ],
        out_specs=_row_spec(H2),
        out_shape=jax.ShapeDtypeStruct((N, H2), jnp.float32),
    )(agg1[0], agg1[1], g1, d0, d1, W2, b1.reshape(1, H1))

    agg2 = _make_agg(H2)(g2, srcp, dstp)[:, :N]

    out = pl.pallas_call(
        _tc3_body,
        grid=grid,
        in_specs=[
            _row_spec(H2),
            _row_spec(H2),
            _row_spec(H2),
            _row_spec(8),
            _row_spec(8),
            _full_spec(1, H2),
            _full_spec(1, H2),
        ],
        out_specs=_row_spec(8),
        out_shape=jax.ShapeDtypeStruct((N, 8), jnp.float32),
    )(agg2[0], agg2[1], g2, d0, d1, Wfc.reshape(1, H2), b2.reshape(1, H2))

    return out[:, 0] + bfc[0]
